# Initial kernel scaffold; baseline (speedup 1.0000x reference)
import jax
import jax.numpy as jnp
from jax.experimental import pallas as pl


def kernel(input_ids, W_word, W_pos):
    raise NotImplementedError("write your pallas kernel here")


# speedup vs baseline: 1.0667x; 1.0667x over previous
import functools

import jax
import jax.numpy as jnp
from jax import lax
from jax.experimental import pallas as pl
from jax.experimental.pallas import tpu as pltpu
from jax.experimental.pallas import tpu_sc as plsc

_VOCAB = 50304
_MAX_POS = 8192
_EMBED = 768
_BATCH = 4
_SEQ = 8192

_NC = 2
_NS = 16
_NW = _NC * _NS
_POS_PER_W = _SEQ // _NW
_C = 32
_NCHUNK = _POS_PER_W // _C
_NVREG = _EMBED // 16


def _body(ids_hbm, w_word, w_pos, out_hbm, idx_v, rbuf, posbuf, sem_g):
    cid = lax.axis_index("c")
    sid = lax.axis_index("s")
    wid = sid * _NC + cid
    pos0 = wid * _POS_PER_W

    pltpu.sync_copy(ids_hbm.at[:, pl.ds(pos0, _POS_PER_W)], idx_v)

    def chunk_body(g, _):
        cbase = pos0 + g * _C
        pltpu.sync_copy(w_pos.at[pl.ds(cbase, _C)], posbuf)
        for b in range(_BATCH):
            pltpu.async_copy(
                w_word.at[idx_v.at[b, pl.ds(g * _C, _C)]], rbuf, sem_g
            ).wait()

            def add_row(r, carry):
                for j in range(_NVREG):
                    s = pl.ds(j * 16, 16)
                    rbuf[r, s] = rbuf[r, s] + posbuf[r, s]
                return carry

            lax.fori_loop(0, _C, add_row, 0, unroll=False)
            pltpu.sync_copy(rbuf, out_hbm.at[pl.ds(b * _SEQ + cbase, _C)])
        return _

    lax.fori_loop(0, _NCHUNK, chunk_body, 0, unroll=False)


@jax.jit
def _embed(input_ids, w_word, w_pos):
    mesh = plsc.VectorSubcoreMesh(core_axis_name="c", subcore_axis_name="s")
    k = pl.kernel(
        _body,
        out_type=jax.ShapeDtypeStruct((_BATCH * _SEQ, _EMBED), jnp.float32),
        mesh=mesh,
        scratch_types=[
            pltpu.VMEM((_BATCH, _POS_PER_W), jnp.int32),
            pltpu.VMEM((_C, _EMBED), jnp.float32),
            pltpu.VMEM((_C, _EMBED), jnp.float32),
            pltpu.SemaphoreType.DMA,
        ],
    )
    return k(input_ids, w_word, w_pos)


def kernel(input_ids, W_word, W_pos):
    ids = input_ids.astype(jnp.int32)
    out = _embed(ids, W_word, W_pos)
    return out.reshape(_BATCH, _SEQ, _EMBED)
